# Initial kernel scaffold; baseline (speedup 1.0000x reference)
import jax
import jax.numpy as jnp
from jax.experimental import pallas as pl


def kernel(x, x_c, x_s, table, w, b):
    raise NotImplementedError("write your pallas kernel here")


# speedup vs baseline: 2.2359x; 2.2359x over previous
import functools

import jax
import jax.numpy as jnp
from jax import lax
from jax.experimental import pallas as pl
from jax.experimental.pallas import tpu as pltpu
from jax.experimental.pallas import tpu_sc as plsc

N_ROWS = 1_000_000
H = 64
BATCH = 4096
HIST = 50
N_PER = BATCH * HIST
TOT = 3 * N_PER

MV_BLK = 8192
MV_GRID = (N_ROWS + MV_BLK - 1) // MV_BLK
T_PAD = MV_GRID * MV_BLK


def _mv_body(tab_ref, w_ref, b_ref, t_ref):
    tab_t = tab_ref[...].T
    t_ref[...] = jnp.sum(tab_t * w_ref[...], axis=0) + b_ref[0]


def _project_table(table, w_col, b):
    return pl.pallas_call(
        _mv_body,
        grid=(MV_GRID,),
        in_specs=[
            pl.BlockSpec((MV_BLK, H), lambda i: (i, 0)),
            pl.BlockSpec((H, 1), lambda i: (0, 0)),
            pl.BlockSpec(memory_space=pltpu.SMEM),
        ],
        out_specs=pl.BlockSpec((MV_BLK,), lambda i: (i,)),
        out_shape=jax.ShapeDtypeStruct((T_PAD,), jnp.float32),
    )(table, w_col, b)


NC, NS = 2, 16
NW = NC * NS
WIN = 128
ROWS = 152
PER_W = ROWS * WIN
TOT_PAD = NW * PER_W
LAG = 8


def _gather(t, idx3):
    mesh = plsc.VectorSubcoreMesh(core_axis_name="c", subcore_axis_name="s")

    @functools.partial(
        pl.kernel,
        mesh=mesh,
        out_type=jax.ShapeDtypeStruct((NW, ROWS, WIN), jnp.float32),
        scratch_types=[
            pltpu.VMEM((ROWS, WIN), jnp.int32),
            pltpu.VMEM((ROWS, WIN), jnp.float32),
            pltpu.SemaphoreType.DMA,
            pltpu.SemaphoreType.DMA,
        ],
    )
    def sc_gather(t_hbm, idx_hbm, out_hbm, idx_v, buf_v, sem_io, sem_g):
        wid = lax.axis_index("s") * NC + lax.axis_index("c")
        pltpu.async_copy(idx_hbm.at[wid], idx_v, sem_io).wait()

        def gather_row(j):
            return pltpu.make_async_copy(
                t_hbm.at[idx_v.at[j]], buf_v.at[j], sem_g)

        def body(j, carry):
            gather_row(j).start()

            @pl.when(j >= LAG)
            def _():
                gather_row(0).wait()

            return carry

        lax.fori_loop(0, ROWS, body, 0)
        for _ in range(LAG):
            gather_row(0).wait()

        pltpu.async_copy(buf_v, out_hbm.at[wid], sem_io).wait()

    return sc_gather(t, idx3)


def kernel(x, x_c, x_s, table, w, b):
    t = _project_table(table, w, b)

    idx = jnp.concatenate(
        [x.reshape(-1), x_c.reshape(-1), x_s.reshape(-1)]).astype(jnp.int32)
    idx = jnp.concatenate(
        [idx, jnp.zeros((TOT_PAD - TOT,), jnp.int32)]).reshape(NW, ROWS, WIN)

    vals = _gather(t, idx)
    flat = vals.reshape(TOT_PAD)[:TOT]
    y = flat[:N_PER].reshape(BATCH, HIST, 1)
    y_c = flat[N_PER:2 * N_PER].reshape(BATCH, HIST, 1)
    y_s = flat[2 * N_PER:].reshape(BATCH, HIST, 1)
    return (y, y_c, y_s)
